# Initial kernel scaffold; baseline (speedup 1.0000x reference)
import jax
import jax.numpy as jnp
from jax.experimental import pallas as pl


def kernel(x, hx, tree_ids_d, tree_ids_dr, tree_ids_dl, W_ioux, b_ioux, W_iouh0, W_iouh1, W_fx, b_fx, W_fh0, W_fh1, W_fh2, W_fh3):
    raise NotImplementedError("write your pallas kernel here")


# speedup vs baseline: 9.0850x; 9.0850x over previous
import functools

import jax
import jax.numpy as jnp
from jax import lax
from jax.experimental import pallas as pl
from jax.experimental.pallas import tpu as pltpu
from jax.experimental.pallas import tpu_sc as plsc

_F32 = jnp.float32

_NC = 2
_NS = 16
_LN = 16
_NW = _NC * _NS
_CHUNK = 128


def _mm(a, b):
    return lax.dot_general(a, b, (((1,), (0,)), ((), ())),
                           preferred_element_type=_F32)


def _mmT(a, b):
    return lax.dot_general(a, b, (((0,), (0,)), ((), ())),
                           preferred_element_type=_F32)


def _sc_stage_factory(B, L, H):
    n_ch = L // _CHUNK
    rows_per_w = B // _NW
    mesh = plsc.VectorSubcoreMesh(core_axis_name="c", subcore_axis_name="s")

    @functools.partial(
        pl.kernel,
        out_type=[jax.ShapeDtypeStruct((B * L, H), _F32)] * 5,
        mesh=mesh,
        scratch_types=[
            pltpu.VMEM((n_ch, _CHUNK), jnp.int32),
            pltpu.VMEM((n_ch, _CHUNK), jnp.int32),
            pltpu.VMEM((n_ch, _CHUNK), jnp.int32),
            pltpu.VMEM((n_ch, _CHUNK), jnp.int32),
            pltpu.VMEM((n_ch, _CHUNK), jnp.int32),
            pltpu.VMEM((n_ch, _CHUNK), jnp.int32),
            pltpu.VMEM((n_ch, _CHUNK), jnp.int32),
            pltpu.VMEM((n_ch, _CHUNK), jnp.int32),
            pltpu.VMEM((_CHUNK, H), _F32),
            pltpu.VMEM((_CHUNK, H), _F32),
            pltpu.VMEM_SHARED((_NS * L, H), _F32),
            pltpu.SemaphoreType.DMA,
        ],
    )
    def sc_stage(x_hbm, h0_hbm, idxd_hbm, idxr_hbm, idxl_hbm,
                 xgd_hbm, h0gr_hbm, h0gl_hbm, h0sr_hbm, h0sl_hbm,
                 lidx_d, lidx_r, lidx_l, gidx_d, gidx_r, gidx_l,
                 sidx_r, sidx_l, buf, zbuf, acc, sem):
        cid = lax.axis_index("c")
        sid = lax.axis_index("s")
        w = sid * _NC + cid
        accbase = sid * L

        def zfill(j, carry):
            for k in range(H // _LN):
                zbuf[j, pl.ds(k * _LN, _LN)] = jnp.zeros((_LN,), _F32)
            return carry
        lax.fori_loop(0, _CHUNK, zfill, 0)

        def zero_acc():
            for ch in range(n_ch):
                pltpu.sync_copy(
                    zbuf, acc.at[pl.ds(accbase + ch * _CHUNK, _CHUNK)])

        for t in range(rows_per_w):
            b = w * rows_per_w + t
            bL = b * L

            pltpu.sync_copy(idxd_hbm.at[b], lidx_d)
            pltpu.sync_copy(idxr_hbm.at[b], lidx_r)
            pltpu.sync_copy(idxl_hbm.at[b], lidx_l)
            for li, gi, off in ((lidx_d, gidx_d, bL), (lidx_r, gidx_r, bL),
                                (lidx_l, gidx_l, bL), (lidx_r, sidx_r, accbase),
                                (lidx_l, sidx_l, accbase)):
                for j in range(n_ch):
                    for k in range(_CHUNK // _LN):
                        sl = pl.ds(k * _LN, _LN)
                        gi[j, sl] = li[j, sl] + off

            for gi, src, dst in ((gidx_d, x_hbm, xgd_hbm),
                                 (gidx_r, h0_hbm, h0gr_hbm),
                                 (gidx_l, h0_hbm, h0gl_hbm)):
                for ch in range(n_ch):
                    pltpu.async_copy(src.at[gi.at[ch]], buf, sem).wait()
                    pltpu.sync_copy(
                        buf, dst.at[pl.ds(bL + ch * _CHUNK, _CHUNK)])

            for si, dst in ((sidx_r, h0sr_hbm), (sidx_l, h0sl_hbm)):
                zero_acc()
                for ch in range(n_ch):
                    pltpu.sync_copy(
                        h0_hbm.at[pl.ds(bL + ch * _CHUNK, _CHUNK)], buf)
                    pltpu.sync_copy(buf, acc.at[si.at[ch]], add=True)
                pltpu.sync_copy(acc.at[pl.ds(accbase, L)],
                                dst.at[pl.ds(bL, L)])

    return sc_stage


def _cell_body(idx_d_ref, x_ref, h0_ref, c0_ref,
               xgd_ref, h0gr_ref, h0gl_ref, h0sr_ref, h0sl_ref,
               W_ioux_ref, b_ioux_ref, W_iouh0_ref, W_iouh1_ref,
               W_fx_ref, b_fx_ref, W_fh0_ref, W_fh1_ref, W_fh2_ref, W_fh3_ref,
               h_out_ref, c_out_ref):
    L = x_ref.shape[1]
    H = W_fx_ref.shape[1]

    x = x_ref[0]
    h0 = h0_ref[0]
    c0 = c0_ref[0]
    idx_d = idx_d_ref[0]

    iou = (_mm(x, W_ioux_ref[...]) + b_ioux_ref[...]
           + _mm(h0sr_ref[0], W_iouh0_ref[...])
           + _mm(h0sl_ref[0], W_iouh1_ref[...]))
    i = jax.nn.sigmoid(iou[:, :H])
    o = jax.nn.sigmoid(iou[:, H:2 * H])
    u = jnp.tanh(iou[:, 2 * H:])

    f = jax.nn.sigmoid(_mm(xgd_ref[0], W_fx_ref[...]) + b_fx_ref[...]
                       + _mm(h0gr_ref[0], W_fh0_ref[...] + W_fh1_ref[...])
                       + _mm(h0gl_ref[0], W_fh2_ref[...] + W_fh3_ref[...]))

    row_iota = lax.broadcasted_iota(jnp.int32, (L, L), 0)
    T_d = (row_iota == idx_d).astype(_F32)

    c = i * u + _mm(T_d, f * c0)

    counts = jnp.sum(T_d, axis=1, keepdims=True)
    kpos = lax.broadcasted_iota(jnp.int32, (L, 1), 0)
    upd = (counts > 0.0) & (kpos != 0)

    h = o * jnp.tanh(c)
    h_out_ref[0] = jnp.where(upd, h, h0)
    c_out_ref[0] = jnp.where(upd, c, c0)


def kernel(x, hx, tree_ids_d, tree_ids_dr, tree_ids_dl, W_ioux, b_ioux,
           W_iouh0, W_iouh1, W_fx, b_fx, W_fh0, W_fh1, W_fh2, W_fh3):
    B, L, E = x.shape
    H = W_fx.shape[1]
    n_ch = L // _CHUNK
    h0, c0 = hx[0], hx[1]

    idx_d3 = tree_ids_d.astype(jnp.int32).reshape(B, n_ch, _CHUNK)
    idx_r3 = tree_ids_dr.astype(jnp.int32).reshape(B, n_ch, _CHUNK)
    idx_l3 = tree_ids_dl.astype(jnp.int32).reshape(B, n_ch, _CHUNK)

    sc_stage = _sc_stage_factory(B, L, H)
    xgd, h0gr, h0gl, h0sr, h0sl = sc_stage(
        x.reshape(B * L, E), h0.reshape(B * L, H), idx_d3, idx_r3, idx_l3)

    idx_d = tree_ids_d.astype(jnp.int32).reshape(B, 1, L)
    b_ioux2 = b_ioux.reshape(1, 3 * H)
    b_fx2 = b_fx.reshape(1, H)

    def row_spec(shape):
        nd = len(shape)
        return pl.BlockSpec((1,) + shape[1:], lambda b: (b,) + (0,) * (nd - 1))

    def full_spec(shape):
        nd = len(shape)
        return pl.BlockSpec(shape, lambda b: (0,) * nd)

    h_out, c_out = pl.pallas_call(
        _cell_body,
        grid=(B,),
        in_specs=[
            row_spec((B, 1, L)),
            row_spec((B, L, E)), row_spec((B, L, H)), row_spec((B, L, H)),
            row_spec((B, L, H)), row_spec((B, L, H)), row_spec((B, L, H)),
            row_spec((B, L, H)), row_spec((B, L, H)),
            full_spec((E, 3 * H)), full_spec((1, 3 * H)),
            full_spec((H, 3 * H)), full_spec((H, 3 * H)),
            full_spec((E, H)), full_spec((1, H)),
            full_spec((H, H)), full_spec((H, H)),
            full_spec((H, H)), full_spec((H, H)),
        ],
        out_specs=[row_spec((B, L, H)), row_spec((B, L, H))],
        out_shape=[jax.ShapeDtypeStruct((B, L, H), _F32),
                   jax.ShapeDtypeStruct((B, L, H), _F32)],
    )(idx_d, x, h0, c0,
      xgd.reshape(B, L, H), h0gr.reshape(B, L, H), h0gl.reshape(B, L, H),
      h0sr.reshape(B, L, H), h0sl.reshape(B, L, H),
      W_ioux, b_ioux2, W_iouh0, W_iouh1,
      W_fx, b_fx2, W_fh0, W_fh1, W_fh2, W_fh3)
    return (h_out, c_out)


# speedup vs baseline: 10.2809x; 1.1316x over previous
import functools

import jax
import jax.numpy as jnp
from jax import lax
from jax.experimental import pallas as pl
from jax.experimental.pallas import tpu as pltpu
from jax.experimental.pallas import tpu_sc as plsc

_F32 = jnp.float32

_NC = 2
_NS = 16
_LN = 16
_NW = _NC * _NS
_CHUNK = 128


def _mm(a, b):
    return lax.dot_general(a, b, (((1,), (0,)), ((), ())),
                           preferred_element_type=_F32)


def _mmT(a, b):
    return lax.dot_general(a, b, (((0,), (0,)), ((), ())),
                           preferred_element_type=_F32)


def _sc_stage_factory(B, L, H):
    n_ch = L // _CHUNK
    half = L // 2
    rows_per_w = B // _NW
    mesh = plsc.VectorSubcoreMesh(core_axis_name="c", subcore_axis_name="s")

    @functools.partial(
        pl.kernel,
        out_type=[jax.ShapeDtypeStruct((B * L, H), _F32)] * 5,
        mesh=mesh,
        scratch_types=[
            pltpu.VMEM((n_ch, _CHUNK), jnp.int32),
            pltpu.VMEM((n_ch, _CHUNK), jnp.int32),
            pltpu.VMEM((n_ch, _CHUNK), jnp.int32),
            pltpu.VMEM((n_ch, _CHUNK), jnp.int32),
            pltpu.VMEM((n_ch, _CHUNK), jnp.int32),
            pltpu.VMEM((_CHUNK, H), _F32),
            pltpu.VMEM((_CHUNK, H), _F32),
            pltpu.VMEM((_CHUNK, H), _F32),
            pltpu.VMEM_SHARED((_NS * L, H), _F32),
            pltpu.SemaphoreType.DMA,
            pltpu.SemaphoreType.DMA,
            pltpu.SemaphoreType.DMA,
            pltpu.SemaphoreType.DMA,
            pltpu.SemaphoreType.DMA,
            pltpu.SemaphoreType.DMA,
            pltpu.SemaphoreType.DMA,
            pltpu.SemaphoreType.DMA,
        ],
    )
    def sc_stage(x_hbm, h0_hbm, idxd_hbm, idxr_hbm, idxl_hbm,
                 xgd_hbm, h0gr_hbm, h0gl_hbm, h0sr_hbm, h0sl_hbm,
                 gidx_d, gidx_r, gidx_l, sidx_r, sidx_l,
                 G0, G1, zbuf, acc,
                 isem, gsemA, gsemB, wsemA, wsemB, zsem, ssemA, ssemB):
        cid = lax.axis_index("c")
        sid = lax.axis_index("s")
        w = sid * _NC + cid
        accbase = sid * L
        G = (G0, G1)
        gsem = (gsemA, gsemB)
        wsem = (wsemA, wsemB)
        ssem = (ssemA, ssemB)

        def zfill(j, carry):
            for k in range(H // _LN):
                zbuf[j, pl.ds(k * _LN, _LN)] = jnp.zeros((_LN,), _F32)
            return carry
        lax.fori_loop(0, _CHUNK, zfill, 0)

        def row_body(t, carry):
            b = w * rows_per_w + t
            bL = b * L

            i0 = pltpu.async_copy(idxd_hbm.at[b], gidx_d, isem)
            i1 = pltpu.async_copy(idxr_hbm.at[b], gidx_r, isem)
            i2 = pltpu.async_copy(idxl_hbm.at[b], gidx_l, isem)
            i0.wait(); i1.wait(); i2.wait()
            for j in range(n_ch):
                for k in range(_CHUNK // _LN):
                    sl = pl.ds(k * _LN, _LN)
                    sidx_r[j, sl] = gidx_r[j, sl] + accbase
                    sidx_l[j, sl] = gidx_l[j, sl] + accbase
                    gidx_d[j, sl] = gidx_d[j, sl] + bL
                    gidx_r[j, sl] = gidx_r[j, sl] + bL
                    gidx_l[j, sl] = gidx_l[j, sl] + bL

            zs = [pltpu.async_copy(
                      zbuf, acc.at[pl.ds(accbase + ch * _CHUNK, _CHUNK)], zsem)
                  for ch in range(n_ch)]
            for z in zs:
                z.wait()

            def scatter(si):
                sds = {}
                ads = {}
                for ch in range(n_ch):
                    bid = ch % 2
                    if ch >= 2:
                        ads[ch - 2].wait()
                    sds[ch] = pltpu.async_copy(
                        h0_hbm.at[pl.ds(bL + ch * _CHUNK, _CHUNK)],
                        G[bid], gsem[bid])
                    sds[ch].wait()
                    ads[ch] = pltpu.async_copy(
                        G[bid], acc.at[si.at[ch]], ssem[bid], add=True)
                ads[n_ch - 2].wait()
                ads[n_ch - 1].wait()

            scatter(sidx_r)
            ro_r = pltpu.async_copy(acc.at[pl.ds(accbase, L)],
                                    h0sr_hbm.at[pl.ds(bL, L)], wsemA)
            ro_r.wait()
            zs = [pltpu.async_copy(
                      zbuf, acc.at[pl.ds(accbase + ch * _CHUNK, _CHUNK)], zsem)
                  for ch in range(n_ch)]
            for z in zs:
                z.wait()
            scatter(sidx_l)
            ro_l = pltpu.async_copy(acc.at[pl.ds(accbase, L)],
                                    h0sl_hbm.at[pl.ds(bL, L)], wsemB)

            units = []
            for gi, src, dst in ((gidx_d, x_hbm, xgd_hbm),
                                 (gidx_r, h0_hbm, h0gr_hbm),
                                 (gidx_l, h0_hbm, h0gl_hbm)):
                for ch in range(n_ch):
                    units.append((gi, src, dst, ch))
            gds = {}
            wds = {}

            def writeout(u):
                gi_u, src_u, dst_u, ch_u = units[u]
                return pltpu.async_copy(
                    G[u % 2],
                    dst_u.at[pl.ds(bL + ch_u * _CHUNK, _CHUNK)], wsem[u % 2])

            for u, (gi, src, dst, ch) in enumerate(units):
                bid = u % 2
                if u >= 2:
                    wds[u - 2].wait()
                gds[u] = pltpu.async_copy(
                    src.at[gi.at[ch]], G[bid], gsem[bid])
                if u >= 1:
                    gds[u - 1].wait()
                    wds[u - 1] = writeout(u - 1)
            last = len(units) - 1
            gds[last].wait()
            wds[last] = writeout(last)
            wds[last - 1].wait()
            wds[last].wait()
            ro_l.wait()
            return carry

        lax.fori_loop(0, rows_per_w, row_body, 0)

    return sc_stage


def _cell_body(idx_d_ref, x_ref, h0_ref, c0_ref,
               xgd_ref, h0gr_ref, h0gl_ref, h0sr_ref, h0sl_ref,
               W_ioux_ref, b_ioux_ref, W_iouh0_ref, W_iouh1_ref,
               W_fx_ref, b_fx_ref, W_fh0_ref, W_fh1_ref, W_fh2_ref, W_fh3_ref,
               h_out_ref, c_out_ref):
    L = x_ref.shape[1]
    H = W_fx_ref.shape[1]

    x = x_ref[0]
    h0 = h0_ref[0]
    c0 = c0_ref[0]
    idx_d = idx_d_ref[0]

    iou = (_mm(x, W_ioux_ref[...]) + b_ioux_ref[...]
           + _mm(h0sr_ref[0], W_iouh0_ref[...])
           + _mm(h0sl_ref[0], W_iouh1_ref[...]))
    i = jax.nn.sigmoid(iou[:, :H])
    o = jax.nn.sigmoid(iou[:, H:2 * H])
    u = jnp.tanh(iou[:, 2 * H:])

    f = jax.nn.sigmoid(_mm(xgd_ref[0], W_fx_ref[...]) + b_fx_ref[...]
                       + _mm(h0gr_ref[0], W_fh0_ref[...] + W_fh1_ref[...])
                       + _mm(h0gl_ref[0], W_fh2_ref[...] + W_fh3_ref[...]))

    row_iota = lax.broadcasted_iota(jnp.int32, (L, L), 0)
    T_d = (row_iota == idx_d).astype(_F32)

    c = i * u + _mm(T_d, f * c0)

    counts = jnp.sum(T_d, axis=1, keepdims=True)
    kpos = lax.broadcasted_iota(jnp.int32, (L, 1), 0)
    upd = (counts > 0.0) & (kpos != 0)

    h = o * jnp.tanh(c)
    h_out_ref[0] = jnp.where(upd, h, h0)
    c_out_ref[0] = jnp.where(upd, c, c0)


def kernel(x, hx, tree_ids_d, tree_ids_dr, tree_ids_dl, W_ioux, b_ioux,
           W_iouh0, W_iouh1, W_fx, b_fx, W_fh0, W_fh1, W_fh2, W_fh3):
    B, L, E = x.shape
    H = W_fx.shape[1]
    n_ch = L // _CHUNK
    h0, c0 = hx[0], hx[1]

    idx_d3 = tree_ids_d.astype(jnp.int32).reshape(B, n_ch, _CHUNK)
    idx_r3 = tree_ids_dr.astype(jnp.int32).reshape(B, n_ch, _CHUNK)
    idx_l3 = tree_ids_dl.astype(jnp.int32).reshape(B, n_ch, _CHUNK)

    sc_stage = _sc_stage_factory(B, L, H)
    xgd, h0gr, h0gl, h0sr, h0sl = sc_stage(
        x.reshape(B * L, E), h0.reshape(B * L, H), idx_d3, idx_r3, idx_l3)

    idx_d = tree_ids_d.astype(jnp.int32).reshape(B, 1, L)
    b_ioux2 = b_ioux.reshape(1, 3 * H)
    b_fx2 = b_fx.reshape(1, H)

    def row_spec(shape):
        nd = len(shape)
        return pl.BlockSpec((1,) + shape[1:], lambda b: (b,) + (0,) * (nd - 1))

    def full_spec(shape):
        nd = len(shape)
        return pl.BlockSpec(shape, lambda b: (0,) * nd)

    h_out, c_out = pl.pallas_call(
        _cell_body,
        grid=(B,),
        in_specs=[
            row_spec((B, 1, L)),
            row_spec((B, L, E)), row_spec((B, L, H)), row_spec((B, L, H)),
            row_spec((B, L, H)), row_spec((B, L, H)), row_spec((B, L, H)),
            row_spec((B, L, H)), row_spec((B, L, H)),
            full_spec((E, 3 * H)), full_spec((1, 3 * H)),
            full_spec((H, 3 * H)), full_spec((H, 3 * H)),
            full_spec((E, H)), full_spec((1, H)),
            full_spec((H, H)), full_spec((H, H)),
            full_spec((H, H)), full_spec((H, H)),
        ],
        out_specs=[row_spec((B, L, H)), row_spec((B, L, H))],
        out_shape=[jax.ShapeDtypeStruct((B, L, H), _F32),
                   jax.ShapeDtypeStruct((B, L, H), _F32)],
    )(idx_d, x, h0, c0,
      xgd.reshape(B, L, H), h0gr.reshape(B, L, H), h0gl.reshape(B, L, H),
      h0sr.reshape(B, L, H), h0sl.reshape(B, L, H),
      W_ioux, b_ioux2, W_iouh0, W_iouh1,
      W_fx, b_fx2, W_fh0, W_fh1, W_fh2, W_fh3)
    return (h_out, c_out)
